# Initial kernel scaffold; baseline (speedup 1.0000x reference)
import jax
import jax.numpy as jnp
from jax.experimental import pallas as pl


def kernel(items, timesteps, item_embedding_table):
    raise NotImplementedError("write your pallas kernel here")


# speedup vs baseline: 4.6422x; 4.6422x over previous
import functools

import jax
import jax.numpy as jnp
from jax import lax
from jax.experimental import pallas as pl
from jax.experimental.pallas import tpu as pltpu
from jax.experimental.pallas import tpu_sc as plsc

VOCAB = 1000
D = 32
N = 4096 * 200

NC = 2
NS = 16
NW = NC * NS
PER_W = N // NW

C = 128
K = 8
CHUNK = C * K
N_ITER = PER_W // CHUNK


@functools.partial(
    pl.kernel,
    out_type=jax.ShapeDtypeStruct((N, D), jnp.float32),
    mesh=plsc.VectorSubcoreMesh(
        core_axis_name="c", subcore_axis_name="s", num_cores=NC, num_subcores=NS
    ),
    scratch_types=[
        pltpu.VMEM((CHUNK,), jnp.int32),
        pltpu.VMEM((CHUNK, D), jnp.float32),
        pltpu.SemaphoreType.DMA,
    ],
    compiler_params=pltpu.CompilerParams(use_tc_tiling_on_sc=False),
)
def _gather_kernel(table_hbm, items_hbm, out_hbm, idx_v, rows_v, sem):
    wid = lax.axis_index("s") * NC + lax.axis_index("c")
    base = wid * PER_W

    def body(i, _):
        off = base + i * CHUNK
        pltpu.sync_copy(items_hbm.at[pl.ds(off, CHUNK)], idx_v)
        descs = [
            pltpu.async_copy(
                table_hbm.at[idx_v.at[pl.ds(j * C, C)]],
                rows_v.at[pl.ds(j * C, C)],
                sem,
            )
            for j in range(K)
        ]
        for d in descs:
            d.wait()
        pltpu.sync_copy(rows_v, out_hbm.at[pl.ds(off, CHUNK)])
        return 0

    lax.fori_loop(0, N_ITER, body, 0)


def kernel(items, timesteps, item_embedding_table):
    del timesteps
    items_flat = items.reshape(-1).astype(jnp.int32)
    out = _gather_kernel(item_embedding_table, items_flat)
    return out.reshape(items.shape + (D,))
